# baseline (speedup 1.0000x reference)
import functools
import math

import jax
import jax.numpy as jnp
from jax import lax
from jax.experimental import pallas as pl
from jax.experimental.pallas import tpu as pltpu
from jax.experimental.pallas import tpu_sc as plsc

_D = 128
_S = 200
_B = 1024
_N = _B * _S

_info = plsc.get_sparse_core_info()
_NC = _info.num_cores
_NS = _info.num_subcores
_NW = _NC * _NS
_NSLICE = 2
_BSL = _B // _NSLICE
_NSL = _N // _NSLICE
_PER_W = _NSL // _NW
_CHUNK = 128
_CHUNKS = _PER_W // _CHUNK
_NBUF = 6
_PREF = 3


def _sc_gather(tok3d, table):
    mesh = plsc.VectorSubcoreMesh(core_axis_name="c", subcore_axis_name="s")

    @functools.partial(
        pl.kernel,
        out_type=jax.ShapeDtypeStruct((_NSL, _D), jnp.float32),
        mesh=mesh,
        scratch_types=[
            pltpu.VMEM((_CHUNKS, _CHUNK), jnp.int32),
            *([pltpu.VMEM((_CHUNK, _D), jnp.float32)] * _NBUF),
            *([pltpu.SemaphoreType.DMA] * _NBUF),
            *([pltpu.SemaphoreType.DMA] * _NBUF),
        ],
    )
    def k(tok_hbm, table_hbm, out_hbm, idx_v, *bufsems):
        rows = bufsems[:_NBUF]
        gsem = bufsems[_NBUF : 2 * _NBUF]
        ssem = bufsems[2 * _NBUF :]
        wid = lax.axis_index("s") * _NC + lax.axis_index("c")
        base = wid * _PER_W
        pltpu.sync_copy(tok_hbm.at[wid], idx_v)

        for j in range(_PREF):
            pltpu.async_copy(table_hbm.at[idx_v.at[j]], rows[j], gsem[j])

        def turn(c, j):
            jn = (j + _PREF) % _NBUF
            pltpu.make_async_copy(
                table_hbm.at[idx_v.at[c]], rows[j], gsem[j]
            ).wait()
            pltpu.async_copy(
                rows[j], out_hbm.at[pl.ds(base + c * _CHUNK, _CHUNK)], ssem[j]
            )

            @pl.when(c + _PREF < _CHUNKS)
            def _():
                old = c + _PREF - _NBUF
                @pl.when(old >= 0)
                def _():
                    pltpu.make_async_copy(
                        rows[jn],
                        out_hbm.at[pl.ds(base + old * _CHUNK, _CHUNK)],
                        ssem[jn],
                    ).wait()

                pltpu.async_copy(
                    table_hbm.at[idx_v.at[c + _PREF]], rows[jn], gsem[jn]
                )

        def step(c, carry):
            for j in range(_NBUF):
                @pl.when(lax.rem(c, _NBUF) == j)
                def _(c=c, j=j):
                    turn(c, j)
            return carry

        lax.fori_loop(0, _CHUNKS, step, 0)

        for j in range(_NBUF):
            pltpu.make_async_copy(
                rows[j], out_hbm.at[pl.ds(0, _CHUNK)], ssem[j]
            ).wait()

    return k(tok3d, table)


_BB = 32


def _tc_body_first(g_ref, pos_ref, w_ref, o_ref):
    scale = math.sqrt(_D)
    x = g_ref[...] + pos_ref[...][None]
    ws = w_ref[...] * scale
    xf = x.reshape(_BB * _S, _D)
    y = lax.dot_general(
        xf, ws, (((1,), (1,)), ((), ())), preferred_element_type=jnp.float32
    )
    o_ref[...] = y.reshape(_BB, _S, _D)


def _tc_body(g_ref, pos_ref, w_ref, acc_ref, o_ref):
    del acc_ref
    _tc_body_first(g_ref, pos_ref, w_ref, o_ref)


def _tc_project(sl, g3d, pos, w, acc):
    off = sl * (_BSL // _BB)
    specs = [
        pl.BlockSpec((_BB, _S, _D), lambda i: (i, 0, 0)),
        pl.BlockSpec((_S, _D), lambda i: (0, 0)),
        pl.BlockSpec((_D, _D), lambda i: (0, 0)),
    ]
    args = (g3d, pos, w)
    body = _tc_body_first
    aliases = {}
    if acc is not None:
        specs.append(pl.BlockSpec(memory_space=pl.ANY))
        args = args + (acc,)
        body = _tc_body
        aliases = {3: 0}
    return pl.pallas_call(
        body,
        grid=(_BSL // _BB,),
        in_specs=specs,
        out_specs=pl.BlockSpec((_BB, _S, _D), lambda i: (off + i, 0, 0)),
        out_shape=jax.ShapeDtypeStruct((_B, _S, _D), jnp.float32),
        input_output_aliases=aliases,
    )(*args)


@jax.jit
def kernel(tokens, emb_weight, pos_weight, proj_weight):
    tok = tokens.astype(jnp.int32).reshape(_NSLICE, _NW, _CHUNKS, _CHUNK)
    gathered = [_sc_gather(tok[sl], emb_weight) for sl in range(_NSLICE)]
    acc = None
    for sl in range(_NSLICE):
        g3d = gathered[sl].reshape(_BSL, _S, _D)
        acc = _tc_project(sl, g3d, pos_weight, proj_weight, acc)
    return acc


# speedup vs baseline: 1.0188x; 1.0188x over previous
import functools
import math

import jax
import jax.numpy as jnp
from jax import lax
from jax.experimental import pallas as pl
from jax.experimental.pallas import tpu as pltpu
from jax.experimental.pallas import tpu_sc as plsc

_D = 128
_S = 200
_B = 1024
_N = _B * _S

_info = plsc.get_sparse_core_info()
_NC = _info.num_cores
_NS = _info.num_subcores
_NW = _NC * _NS
_NSLICE = 2
_BSL = _B // _NSLICE
_NSL = _N // _NSLICE
_PER_W = _NSL // _NW
_CHUNK = 128
_CHUNKS = _PER_W // _CHUNK
_NBUF = 6
_PREF = 3


def _sc_gather(tok3d, table):
    mesh = plsc.VectorSubcoreMesh(core_axis_name="c", subcore_axis_name="s")

    @functools.partial(
        pl.kernel,
        out_type=jax.ShapeDtypeStruct((_NSL, _D), jnp.float32),
        mesh=mesh,
        scratch_types=[
            pltpu.VMEM((_CHUNKS, _CHUNK), jnp.int32),
            *([pltpu.VMEM((_CHUNK, _D), jnp.float32)] * _NBUF),
            *([pltpu.SemaphoreType.DMA] * _NBUF),
            *([pltpu.SemaphoreType.DMA] * _NBUF),
        ],
    )
    def k(tok_hbm, table_hbm, out_hbm, idx_v, *bufsems):
        rows = bufsems[:_NBUF]
        gsem = bufsems[_NBUF : 2 * _NBUF]
        ssem = bufsems[2 * _NBUF :]
        wid = lax.axis_index("s") * _NC + lax.axis_index("c")
        base = wid * _PER_W
        pltpu.sync_copy(tok_hbm.at[wid], idx_v)

        for j in range(_PREF):
            pltpu.async_copy(table_hbm.at[idx_v.at[j]], rows[j], gsem[j])

        def turn(c, j):
            jn = (j + _PREF) % _NBUF
            pltpu.make_async_copy(
                table_hbm.at[idx_v.at[c]], rows[j], gsem[j]
            ).wait()
            pltpu.async_copy(
                rows[j], out_hbm.at[pl.ds(base + c * _CHUNK, _CHUNK)], ssem[j]
            )

            @pl.when(c + _PREF < _CHUNKS)
            def _():
                old = c + _PREF - _NBUF
                @pl.when(old >= 0)
                def _():
                    pltpu.make_async_copy(
                        rows[jn],
                        out_hbm.at[pl.ds(base + old * _CHUNK, _CHUNK)],
                        ssem[jn],
                    ).wait()

                pltpu.async_copy(
                    table_hbm.at[idx_v.at[c + _PREF]], rows[jn], gsem[jn]
                )

        def step(c, carry):
            for j in range(_NBUF):
                @pl.when(lax.rem(c, _NBUF) == j)
                def _(c=c, j=j):
                    turn(c, j)
            return carry

        lax.fori_loop(0, _CHUNKS, step, 0)

        for j in range(_NBUF):
            pltpu.make_async_copy(
                rows[j], out_hbm.at[pl.ds(0, _CHUNK)], ssem[j]
            ).wait()

    return k(tok3d, table)


_BB = 64


def _tc_body_first(g_ref, pos_ref, w_ref, o_ref):
    scale = math.sqrt(_D)
    x = g_ref[...] + pos_ref[...][None]
    ws = w_ref[...] * scale
    xf = x.reshape(_BB * _S, _D)
    y = lax.dot_general(
        xf, ws, (((1,), (1,)), ((), ())), preferred_element_type=jnp.float32
    )
    o_ref[...] = y.reshape(_BB, _S, _D)


def _tc_body(g_ref, pos_ref, w_ref, acc_ref, o_ref):
    del acc_ref
    _tc_body_first(g_ref, pos_ref, w_ref, o_ref)


def _tc_project(sl, g3d, pos, w, acc):
    off = sl * (_BSL // _BB)
    specs = [
        pl.BlockSpec((_BB, _S, _D), lambda i: (i, 0, 0)),
        pl.BlockSpec((_S, _D), lambda i: (0, 0)),
        pl.BlockSpec((_D, _D), lambda i: (0, 0)),
    ]
    args = (g3d, pos, w)
    body = _tc_body_first
    aliases = {}
    if acc is not None:
        specs.append(pl.BlockSpec(memory_space=pl.ANY))
        args = args + (acc,)
        body = _tc_body
        aliases = {3: 0}
    return pl.pallas_call(
        body,
        grid=(_BSL // _BB,),
        in_specs=specs,
        out_specs=pl.BlockSpec((_BB, _S, _D), lambda i: (off + i, 0, 0)),
        out_shape=jax.ShapeDtypeStruct((_B, _S, _D), jnp.float32),
        input_output_aliases=aliases,
    )(*args)


@jax.jit
def kernel(tokens, emb_weight, pos_weight, proj_weight):
    tok = tokens.astype(jnp.int32).reshape(_NSLICE, _NW, _CHUNKS, _CHUNK)
    gathered = [_sc_gather(tok[sl], emb_weight) for sl in range(_NSLICE)]
    acc = None
    for sl in range(_NSLICE):
        g3d = gathered[sl].reshape(_BSL, _S, _D)
        acc = _tc_project(sl, g3d, pos_weight, proj_weight, acc)
    return acc


# speedup vs baseline: 1.0190x; 1.0002x over previous
import functools
import math

import jax
import jax.numpy as jnp
from jax import lax
from jax.experimental import pallas as pl
from jax.experimental.pallas import tpu as pltpu
from jax.experimental.pallas import tpu_sc as plsc

_D = 128
_S = 200
_B = 1024
_N = _B * _S

_info = plsc.get_sparse_core_info()
_NC = _info.num_cores
_NS = _info.num_subcores
_NW = _NC * _NS
_NSLICE = 2
_BSL = _B // _NSLICE
_NSL = _N // _NSLICE
_PER_W = _NSL // _NW
_CHUNK = 128
_CHUNKS = _PER_W // _CHUNK
_NBUF = 6
_PREF = 3


def _sc_gather(tok3d, table):
    mesh = plsc.VectorSubcoreMesh(core_axis_name="c", subcore_axis_name="s")

    @functools.partial(
        pl.kernel,
        out_type=jax.ShapeDtypeStruct((_NSL, _D), jnp.float32),
        mesh=mesh,
        scratch_types=[
            pltpu.VMEM((_CHUNKS, _CHUNK), jnp.int32),
            *([pltpu.VMEM((_CHUNK, _D), jnp.float32)] * _NBUF),
            *([pltpu.SemaphoreType.DMA] * _NBUF),
            *([pltpu.SemaphoreType.DMA] * _NBUF),
        ],
    )
    def k(tok_hbm, table_hbm, out_hbm, idx_v, *bufsems):
        rows = bufsems[:_NBUF]
        gsem = bufsems[_NBUF : 2 * _NBUF]
        ssem = bufsems[2 * _NBUF :]
        wid = lax.axis_index("s") * _NC + lax.axis_index("c")
        base = wid * _PER_W
        pltpu.sync_copy(tok_hbm.at[wid], idx_v)

        for j in range(_PREF):
            pltpu.async_copy(table_hbm.at[idx_v.at[j]], rows[j], gsem[j])

        def turn(c, j):
            jn = (j + _PREF) % _NBUF
            pltpu.make_async_copy(
                table_hbm.at[idx_v.at[c]], rows[j], gsem[j]
            ).wait()
            pltpu.async_copy(
                rows[j], out_hbm.at[pl.ds(base + c * _CHUNK, _CHUNK)], ssem[j]
            )

            @pl.when(c + _PREF < _CHUNKS)
            def _():
                old = c + _PREF - _NBUF
                @pl.when(old >= 0)
                def _():
                    pltpu.make_async_copy(
                        rows[jn],
                        out_hbm.at[pl.ds(base + old * _CHUNK, _CHUNK)],
                        ssem[jn],
                    ).wait()

                pltpu.async_copy(
                    table_hbm.at[idx_v.at[c + _PREF]], rows[jn], gsem[jn]
                )

        def step(c, carry):
            for j in range(_NBUF):
                @pl.when(lax.rem(c, _NBUF) == j)
                def _(c=c, j=j):
                    turn(c, j)
            return carry

        lax.fori_loop(0, _CHUNKS, step, 0)

        for j in range(_NBUF):
            pltpu.make_async_copy(
                rows[j], out_hbm.at[pl.ds(0, _CHUNK)], ssem[j]
            ).wait()

    return k(tok3d, table)


_BB = 64


def _tc_body_first(g_ref, pos_ref, w_ref, o_ref):
    scale = math.sqrt(_D)
    x = g_ref[...] + pos_ref[...][None]
    ws = w_ref[...] * scale
    xf = x.reshape(_BB * _S, _D)
    y = lax.dot_general(
        xf, ws, (((1,), (1,)), ((), ())), preferred_element_type=jnp.float32
    )
    o_ref[...] = y.reshape(_BB, _S, _D)


def _tc_body(g_ref, pos_ref, w_ref, acc_ref, o_ref):
    del acc_ref
    _tc_body_first(g_ref, pos_ref, w_ref, o_ref)


def _tc_project(sl, g3d, pos, w, acc):
    off = sl * (_BSL // _BB)
    specs = [
        pl.BlockSpec((_BB, _S, _D), lambda i: (i, 0, 0)),
        pl.BlockSpec((_S, _D), lambda i: (0, 0)),
        pl.BlockSpec((_D, _D), lambda i: (0, 0)),
    ]
    args = (g3d, pos, w)
    body = _tc_body_first
    aliases = {}
    if acc is not None:
        specs.append(pl.BlockSpec(memory_space=pl.ANY))
        args = args + (acc,)
        body = _tc_body
        aliases = {3: 0}
    return pl.pallas_call(
        body,
        grid=(_BSL // _BB,),
        in_specs=specs,
        out_specs=pl.BlockSpec((_BB, _S, _D), lambda i: (off + i, 0, 0)),
        out_shape=jax.ShapeDtypeStruct((_B, _S, _D), jnp.float32),
        input_output_aliases=aliases,
    )(*args)


@jax.jit
def kernel(tokens, emb_weight, pos_weight, proj_weight):
    tok = tokens.astype(jnp.int32).reshape(_NSLICE, _NW, _CHUNKS, _CHUNK)
    gathered = []
    prev = None
    for sl in range(_NSLICE):
        t = tok[sl]
        if prev is not None:
            t, _ = lax.optimization_barrier((t, prev))
        prev = _sc_gather(t, emb_weight)
        gathered.append(prev)
    acc = None
    for sl in range(_NSLICE):
        g3d = gathered[sl].reshape(_BSL, _S, _D)
        acc = _tc_project(sl, g3d, pos_weight, proj_weight, acc)
    return acc
